# baseline (speedup 1.0000x reference)
import functools
import math

import jax
import jax.numpy as jnp
from jax import lax
from jax.experimental import pallas as pl
from jax.experimental.pallas import tpu as pltpu
from jax.experimental.pallas import tpu_sc as plsc

NUM_EMB = 100000
DIM = 128
_SCALE = math.sqrt(DIM)

NC = 2
NS = 16
NW = NC * NS

BATCH = 4096
SEQ = 50
CHUNK = BATCH // NW
GCH = 16
SPLIT = CHUNK // GCH
N_CHUNKS = SEQ * SPLIT
NB = 10
G = 8


def _make_lookup():
    mesh = plsc.VectorSubcoreMesh(core_axis_name="c", subcore_axis_name="s")

    @functools.partial(
        pl.kernel,
        mesh=mesh,
        out_type=jax.ShapeDtypeStruct((SEQ, BATCH, DIM), jnp.float32),
        scratch_types=[
            pltpu.VMEM((SEQ, CHUNK), jnp.int32),
            pltpu.VMEM((NB, GCH, DIM), jnp.float32),
            pltpu.SemaphoreType.DMA((NB,)),
            pltpu.SemaphoreType.DMA((NB,)),
        ],
    )
    def lookup(idx_hbm, table_hbm, out_hbm, idx_v, rows_v, gsem, osem):
        wid = lax.axis_index("s") * NC + lax.axis_index("c")
        col0 = wid * CHUNK
        pltpu.sync_copy(idx_hbm.at[:, pl.ds(col0, CHUNK)], idx_v)

        def gather(j, b):
            return pltpu.make_async_copy(
                table_hbm.at[idx_v.at[j // SPLIT, pl.ds((j % SPLIT) * GCH, GCH)]],
                rows_v.at[b],
                gsem.at[b],
            )

        def outcp(j, b):
            return pltpu.make_async_copy(
                rows_v.at[b],
                out_hbm.at[j // SPLIT].at[
                    pl.ds(col0 + (j % SPLIT) * GCH, GCH)
                ],
                osem.at[b],
            )

        for b in range(G):
            gather(b, b).start()

        def outer(g, carry):
            for b in range(NB):
                j = g * NB + b
                jn = j + G
                bn = (b + G) % NB

                @pl.when(jn < N_CHUNKS)
                def _():
                    @pl.when(jn >= NB)
                    def _():
                        outcp(jn - NB, bn).wait()

                    gather(jn, bn).start()

                gather(j, b).wait()
                rv = rows_v.at[b]

                def row_body(r, c2):
                    for c in range(DIM // 16):
                        sl = (r, pl.ds(c * 16, 16))
                        rv[sl] = rv[sl] * _SCALE
                    return c2

                lax.fori_loop(0, GCH, row_body, 0)
                outcp(j, b).start()
            return carry

        lax.fori_loop(0, N_CHUNKS // NB, outer, 0)

        for b in range(NB):
            outcp(N_CHUNKS - NB + b, b).wait()

    return lookup


_lookup = _make_lookup()


@jax.jit
def kernel(x, weight):
    out = _lookup(x.T, weight)
    return out.transpose(1, 0, 2)


# speedup vs baseline: 1.0388x; 1.0388x over previous
import functools
import math

import jax
import jax.numpy as jnp
from jax import lax
from jax.experimental import pallas as pl
from jax.experimental.pallas import tpu as pltpu
from jax.experimental.pallas import tpu_sc as plsc

NUM_EMB = 100000
DIM = 128
_SCALE = math.sqrt(DIM)

NC = 2
NS = 16
NW = NC * NS

BATCH = 4096
SEQ = 50
CHUNK = BATCH // NW
GCH = 32
SPLIT = CHUNK // GCH
N_CHUNKS = SEQ * SPLIT
NB = 10
G = 8


def _make_lookup():
    mesh = plsc.VectorSubcoreMesh(core_axis_name="c", subcore_axis_name="s")

    @functools.partial(
        pl.kernel,
        mesh=mesh,
        out_type=jax.ShapeDtypeStruct((SEQ, BATCH, DIM), jnp.float32),
        scratch_types=[
            pltpu.VMEM((SEQ, CHUNK), jnp.int32),
            pltpu.VMEM((NB, GCH, DIM), jnp.float32),
            pltpu.SemaphoreType.DMA((NB,)),
            pltpu.SemaphoreType.DMA((NB,)),
        ],
    )
    def lookup(idx_hbm, table_hbm, out_hbm, idx_v, rows_v, gsem, osem):
        wid = lax.axis_index("s") * NC + lax.axis_index("c")
        col0 = wid * CHUNK
        pltpu.sync_copy(idx_hbm.at[:, pl.ds(col0, CHUNK)], idx_v)

        def gather(j, b):
            return pltpu.make_async_copy(
                table_hbm.at[idx_v.at[j // SPLIT, pl.ds((j % SPLIT) * GCH, GCH)]],
                rows_v.at[b],
                gsem.at[b],
            )

        def outcp(j, b):
            return pltpu.make_async_copy(
                rows_v.at[b],
                out_hbm.at[j // SPLIT].at[
                    pl.ds(col0 + (j % SPLIT) * GCH, GCH)
                ],
                osem.at[b],
            )

        for b in range(G):
            gather(b, b).start()

        def outer(g, carry):
            for b in range(NB):
                j = g * NB + b
                jn = j + G
                bn = (b + G) % NB

                @pl.when(jn < N_CHUNKS)
                def _():
                    @pl.when(jn >= NB)
                    def _():
                        outcp(jn - NB, bn).wait()

                    gather(jn, bn).start()

                gather(j, b).wait()
                rv = rows_v.at[b]

                def row_body(r, c2):
                    for c in range(DIM // 16):
                        sl = (r, pl.ds(c * 16, 16))
                        rv[sl] = rv[sl] * _SCALE
                    return c2

                lax.fori_loop(0, GCH, row_body, 0)
                outcp(j, b).start()
            return carry

        lax.fori_loop(0, N_CHUNKS // NB, outer, 0)

        for b in range(NB):
            outcp(N_CHUNKS - NB + b, b).wait()

    return lookup


_lookup = _make_lookup()


@jax.jit
def kernel(x, weight):
    out = _lookup(x.T, weight)
    return out.transpose(1, 0, 2)
